# baseline (speedup 1.0000x reference)
import jax
import jax.numpy as jnp
from jax.experimental import pallas as pl
from jax.experimental.pallas import tpu as pltpu

_DN_NN = (((1,), (0,)), ((), ()))
_DN_NT = (((1,), (1,)), ((), ()))
_DN_TN = (((0,), (0,)), ((), ()))

_BR = 512
_BRE = 256
_N_STEPS = 20


def _bdot(a, b, dnums):
    return jax.lax.dot_general(a.astype(jnp.bfloat16), b.astype(jnp.bfloat16),
                               dnums, preferred_element_type=jnp.float32)


def _gate(h, p):
    hb = h.astype(jnp.bfloat16).astype(jnp.float32)
    pb = p.astype(jnp.bfloat16).astype(jnp.float32)
    return jnp.sum(hb * pb, axis=1)


def _node_slab(i, t_ref, tb_ref, d_row, adj_slab, hw, b_row):
    trow = t_ref[pl.ds(i * _BR, _BR), :]
    s = trow * d_row
    mult = _bdot(s, tb_ref[...], _DN_NT)
    br, nv = mult.shape
    row = jax.lax.broadcasted_iota(jnp.int32, (br, nv), 0)
    col = jax.lax.broadcasted_iota(jnp.int32, (br, nv), 1)
    m = jnp.where(col == row + i * _BR, 1.0, mult)
    adjusted = m * adj_slab
    return _bdot(adjusted, hw, _DN_NN) + b_row


def _fused_kernel(x_ref, z_ref, adjv_ref, adje_ref, t_ref,
                  w1_ref, b1_ref, p1_ref, w2_ref, b2_ref, p2_ref,
                  w3_ref, b3_ref, p3_ref, out_ref,
                  x1_scr, z2_scr, dn_scr, de_scr, hwn_scr, hwe_scr, tb_ref):
    step = pl.program_id(0)

    @pl.when(step == 0)
    def _prologue1():
        tb_ref[...] = t_ref[...].astype(jnp.bfloat16)
        dn_scr[...] = _gate(z_ref[...], p1_ref[...])[None, :]
        hwn_scr[...] = _bdot(x_ref[...], w1_ref[...], _DN_NN).astype(jnp.bfloat16)

    @pl.when(step < 2)
    def _node1():
        i = step
        res = _node_slab(i, t_ref, tb_ref, dn_scr[...],
                         adjv_ref[pl.ds(i * _BR, _BR), :],
                         hwn_scr[...], b1_ref[...])
        x1_scr[pl.ds(i * _BR, _BR), :] = res

    @pl.when(step == 2)
    def _prologue2():
        de_scr[...] = _gate(x1_scr[...], p2_ref[...])[:, None]
        hwe_scr[...] = _bdot(z_ref[...], w2_ref[...], _DN_NN).astype(jnp.bfloat16)

    @pl.when(jnp.logical_and(step >= 2, step < 18))
    def _edge():
        i = step - 2
        tcol = t_ref[:, pl.ds(i * _BRE, _BRE)]
        s = tcol * de_scr[...]
        mult = _bdot(s, tb_ref[...], _DN_TN)
        br, ne = mult.shape
        row = jax.lax.broadcasted_iota(jnp.int32, (br, ne), 0)
        col = jax.lax.broadcasted_iota(jnp.int32, (br, ne), 1)
        m = jnp.where(col == row + i * _BRE, 1.0, mult)
        adjusted = m * adje_ref[...]
        z2_scr[pl.ds(i * _BRE, _BRE), :] = (
            _bdot(adjusted, hwe_scr[...], _DN_NN) + b2_ref[...])

    @pl.when(step == 18)
    def _prologue3():
        dn_scr[...] = _gate(z2_scr[...], p3_ref[...])[None, :]
        hwn_scr[...] = _bdot(x1_scr[...], w3_ref[...], _DN_NN).astype(jnp.bfloat16)

    @pl.when(step >= 18)
    def _node3():
        i = step - 18
        out_ref[...] = _node_slab(i, t_ref, tb_ref, dn_scr[...],
                                  adjv_ref[pl.ds(i * _BR, _BR), :],
                                  hwn_scr[...], b3_ref[...])


def kernel(X, Z, adj_e, adj_v, T, W1, b1, p1, W2, b2, p2, W3, b3, p3):
    n_v, n_e = T.shape

    def _const(shape):
        return pl.BlockSpec(shape, lambda s: (0, 0))

    return pl.pallas_call(
        _fused_kernel,
        grid=(_N_STEPS,),
        in_specs=[
            _const((n_v, X.shape[1])),
            _const((n_e, Z.shape[1])),
            _const((n_v, n_v)),
            pl.BlockSpec((_BRE, n_e), lambda s: (jnp.clip(s - 2, 0, 15), 0)),
            _const((n_v, n_e)),
            _const(W1.shape), _const((1, b1.shape[0])), _const(p1.shape),
            _const(W2.shape), _const((1, b2.shape[0])), _const(p2.shape),
            _const(W3.shape), _const((1, b3.shape[0])), _const(p3.shape),
        ],
        out_specs=pl.BlockSpec((_BR, W3.shape[1]),
                               lambda s: (jnp.maximum(s - 18, 0), 0)),
        out_shape=jax.ShapeDtypeStruct((n_v, W3.shape[1]), jnp.float32),
        scratch_shapes=[
            pltpu.VMEM((n_v, W1.shape[1]), jnp.float32),
            pltpu.VMEM((n_e, W2.shape[1]), jnp.float32),
            pltpu.VMEM((1, n_e), jnp.float32),
            pltpu.VMEM((n_v, 1), jnp.float32),
            pltpu.VMEM((n_v, W1.shape[1]), jnp.bfloat16),
            pltpu.VMEM((n_e, W2.shape[1]), jnp.bfloat16),
            pltpu.VMEM((n_v, n_e), jnp.bfloat16),
        ],
    )(X, Z, adj_v, adj_e, T,
      W1, b1.reshape(1, -1), p1, W2, b2.reshape(1, -1), p2,
      W3, b3.reshape(1, -1), p3)


# speedup vs baseline: 1.0654x; 1.0654x over previous
import jax
import jax.numpy as jnp
from jax.experimental import pallas as pl
from jax.experimental.pallas import tpu as pltpu

_DN_NN = (((1,), (0,)), ((), ()))
_DN_NT = (((1,), (1,)), ((), ()))
_DN_TN = (((0,), (0,)), ((), ()))

_BR = 512
_N_STEPS = 12


def _bdot(a, b, dnums):
    return jax.lax.dot_general(a.astype(jnp.bfloat16), b.astype(jnp.bfloat16),
                               dnums, preferred_element_type=jnp.float32)


def _gate(h, p):
    hb = h.astype(jnp.bfloat16).astype(jnp.float32)
    pb = p.astype(jnp.bfloat16).astype(jnp.float32)
    return jnp.sum(hb * pb, axis=1)


def _node_slab(i, t_ref, tb_ref, d_row, adj_slab, hw, b_row):
    trow = t_ref[pl.ds(i * _BR, _BR), :]
    s = trow * d_row
    mult = _bdot(s, tb_ref[...], _DN_NT)
    br, nv = mult.shape
    row = jax.lax.broadcasted_iota(jnp.int32, (br, nv), 0)
    col = jax.lax.broadcasted_iota(jnp.int32, (br, nv), 1)
    m = jnp.where(col == row + i * _BR, 1.0, mult)
    adjusted = m * adj_slab
    return _bdot(adjusted, hw, _DN_NN) + b_row


def _fused_kernel(x_ref, z_ref, adjv_ref, adje_ref, t_ref,
                  w1_ref, b1_ref, p1_ref, w2_ref, b2_ref, p2_ref,
                  w3_ref, b3_ref, p3_ref, out_ref,
                  x1_scr, z2_scr, dn_scr, de_scr, hwn_scr, hwe_scr, tb_ref):
    step = pl.program_id(0)

    @pl.when(step == 0)
    def _prologue1():
        tb_ref[...] = t_ref[...].astype(jnp.bfloat16)
        dn_scr[...] = _gate(z_ref[...], p1_ref[...])[None, :]
        hwn_scr[...] = _bdot(x_ref[...], w1_ref[...], _DN_NN).astype(jnp.bfloat16)

    @pl.when(step < 2)
    def _node1():
        i = step
        res = _node_slab(i, t_ref, tb_ref, dn_scr[...],
                         adjv_ref[pl.ds(i * _BR, _BR), :],
                         hwn_scr[...], b1_ref[...])
        x1_scr[pl.ds(i * _BR, _BR), :] = res

    @pl.when(step == 2)
    def _prologue2():
        de_scr[...] = _gate(x1_scr[...], p2_ref[...])[:, None]
        hwe_scr[...] = _bdot(z_ref[...], w2_ref[...], _DN_NN).astype(jnp.bfloat16)

    @pl.when(jnp.logical_and(step >= 2, step < 10))
    def _edge():
        i = step - 2
        tcol = t_ref[:, pl.ds(i * _BR, _BR)]
        s = tcol * de_scr[...]
        mult = _bdot(s, tb_ref[...], _DN_TN)
        br, ne = mult.shape
        row = jax.lax.broadcasted_iota(jnp.int32, (br, ne), 0)
        col = jax.lax.broadcasted_iota(jnp.int32, (br, ne), 1)
        m = jnp.where(col == row + i * _BR, 1.0, mult)
        adjusted = m * adje_ref[...]
        z2_scr[pl.ds(i * _BR, _BR), :] = (
            _bdot(adjusted, hwe_scr[...], _DN_NN) + b2_ref[...])

    @pl.when(step == 10)
    def _prologue3():
        dn_scr[...] = _gate(z2_scr[...], p3_ref[...])[None, :]
        hwn_scr[...] = _bdot(x1_scr[...], w3_ref[...], _DN_NN).astype(jnp.bfloat16)

    @pl.when(step >= 10)
    def _node3():
        i = step - 10
        out_ref[...] = _node_slab(i, t_ref, tb_ref, dn_scr[...],
                                  adjv_ref[pl.ds(i * _BR, _BR), :],
                                  hwn_scr[...], b3_ref[...])


def kernel(X, Z, adj_e, adj_v, T, W1, b1, p1, W2, b2, p2, W3, b3, p3):
    n_v, n_e = T.shape

    def _const(shape):
        return pl.BlockSpec(shape, lambda s: (0, 0))

    return pl.pallas_call(
        _fused_kernel,
        grid=(_N_STEPS,),
        in_specs=[
            _const((n_v, X.shape[1])),
            _const((n_e, Z.shape[1])),
            _const((n_v, n_v)),
            pl.BlockSpec((_BR, n_e), lambda s: (jnp.clip(s - 2, 0, 7), 0)),
            _const((n_v, n_e)),
            _const(W1.shape), _const((1, b1.shape[0])), _const(p1.shape),
            _const(W2.shape), _const((1, b2.shape[0])), _const(p2.shape),
            _const(W3.shape), _const((1, b3.shape[0])), _const(p3.shape),
        ],
        out_specs=pl.BlockSpec((_BR, W3.shape[1]),
                               lambda s: (jnp.maximum(s - 10, 0), 0)),
        out_shape=jax.ShapeDtypeStruct((n_v, W3.shape[1]), jnp.float32),
        scratch_shapes=[
            pltpu.VMEM((n_v, W1.shape[1]), jnp.float32),
            pltpu.VMEM((n_e, W2.shape[1]), jnp.float32),
            pltpu.VMEM((1, n_e), jnp.float32),
            pltpu.VMEM((n_v, 1), jnp.float32),
            pltpu.VMEM((n_v, W1.shape[1]), jnp.bfloat16),
            pltpu.VMEM((n_e, W2.shape[1]), jnp.bfloat16),
            pltpu.VMEM((n_v, n_e), jnp.bfloat16),
        ],
    )(X, Z, adj_v, adj_e, T,
      W1, b1.reshape(1, -1), p1, W2, b2.reshape(1, -1), p2,
      W3, b3.reshape(1, -1), p3)
